# baseline (speedup 1.0000x reference)
import functools

import jax
import jax.numpy as jnp
from jax import lax
from jax.experimental import pallas as pl
from jax.experimental.pallas import tpu as pltpu
from jax.experimental.pallas import tpu_sc as plsc

NC = 2
NS = 16
NW = NC * NS

IDX_ROW = 128
ROWS_PER_CHUNK = 512
K = ROWS_PER_CHUNK // IDX_ROW


def _sc_gather(table, ids3, n_rows_per_w, d):
    n_idx_rows = ids3.shape[1]
    n_chunks = n_rows_per_w // ROWS_PER_CHUNK
    mesh = plsc.VectorSubcoreMesh(
        core_axis_name="c", subcore_axis_name="s", num_cores=NC,
        num_subcores=NS)

    @functools.partial(
        pl.kernel,
        mesh=mesh,
        compiler_params=pltpu.CompilerParams(use_tc_tiling_on_sc=False),
        out_type=jax.ShapeDtypeStruct((NW * n_rows_per_w, d), jnp.float32),
        scratch_types=[
            pltpu.VMEM((n_idx_rows, IDX_ROW), jnp.int32),
            pltpu.VMEM((ROWS_PER_CHUNK, d), jnp.float32),
            pltpu.VMEM((ROWS_PER_CHUNK, d), jnp.float32),
            pltpu.SemaphoreType.DMA,
            pltpu.SemaphoreType.DMA,
            pltpu.SemaphoreType.DMA,
            pltpu.SemaphoreType.DMA,
        ],
    )
    def k(table_hbm, idx_hbm, out_hbm, idx_v, r0, r1, g0, g1, w0, w1):
        wid = lax.axis_index("s") * NC + lax.axis_index("c")
        pltpu.sync_copy(idx_hbm.at[wid], idx_v)
        base = wid * n_rows_per_w

        def fire(c, rbuf, gsem):
            for j in range(K):
                pltpu.async_copy(
                    table_hbm.at[idx_v.at[c * K + j]],
                    rbuf.at[pl.ds(j * IDX_ROW, IDX_ROW)], gsem)

        def wait_g(rbuf, gsem):
            for j in range(K):
                pltpu.make_async_copy(
                    table_hbm.at[idx_v.at[j]],
                    rbuf.at[pl.ds(j * IDX_ROW, IDX_ROW)], gsem).wait()

        def write(c, rbuf, wsem):
            pltpu.async_copy(
                rbuf, out_hbm.at[pl.ds(base + c * ROWS_PER_CHUNK,
                                       ROWS_PER_CHUNK)], wsem)

        def wait_w(rbuf, wsem):
            pltpu.make_async_copy(
                rbuf, out_hbm.at[pl.ds(base, ROWS_PER_CHUNK)], wsem).wait()

        bufs = ((r0, g0, w0), (r1, g1, w1))
        fire(0, r0, g0)
        fire(1, r1, g1)

        def body(c2, carry):
            for b, (rb, gs, ws) in enumerate(bufs):
                c = 2 * c2 + b
                wait_g(rb, gs)
                write(c, rb, ws)
                wait_w(rb, ws)

                @pl.when(c + 2 < n_chunks)
                def _():
                    fire(c + 2, rb, gs)

            return carry

        lax.fori_loop(0, n_chunks // 2, body, 0)

    return k(table, ids3)


def kernel(token_ids, embedding_table):
    b, s = token_ids.shape
    v, d = embedding_table.shape
    n = b * s
    assert n % (NW * ROWS_PER_CHUNK) == 0
    n_rows_per_w = n // NW
    ids3 = token_ids.reshape(NW, n_rows_per_w // IDX_ROW, IDX_ROW)
    ids3 = ids3.astype(jnp.int32)
    out = _sc_gather(embedding_table, ids3, n_rows_per_w, d)
    return out.reshape(b, s, d)


# speedup vs baseline: 1.4057x; 1.4057x over previous
import functools

import jax
import jax.numpy as jnp
from jax import lax
from jax.experimental import pallas as pl
from jax.experimental.pallas import tpu as pltpu
from jax.experimental.pallas import tpu_sc as plsc

NC = 2
NS = 16
NW = NC * NS


def _sc_gather(table, ids_flat, b, s, d):
    v = table.shape[0]
    batches_per_w = b // NW
    ids_per_w = batches_per_w * s
    mesh = plsc.VectorSubcoreMesh(
        core_axis_name="c", subcore_axis_name="s", num_cores=NC,
        num_subcores=NS)

    @functools.partial(
        pl.kernel,
        mesh=mesh,
        compiler_params=pltpu.CompilerParams(use_tc_tiling_on_sc=False),
        out_type=jax.ShapeDtypeStruct((b, s, d), jnp.float32),
        scratch_types=[
            pltpu.VMEM((ids_per_w,), jnp.int32),
            pltpu.VMEM((s, d), jnp.float32),
            pltpu.VMEM((s, d), jnp.float32),
            pltpu.VMEM_SHARED((v, d), jnp.float32),
            pltpu.SemaphoreType.DMA,
            pltpu.SemaphoreType.DMA,
            pltpu.SemaphoreType.DMA,
            pltpu.SemaphoreType.DMA,
        ],
    )
    def k(table_hbm, idx_hbm, out_hbm, idx_v, r0, r1, tab_sh,
          g0, g1, w0, w1):
        sid = lax.axis_index("s")
        wid = sid * NC + lax.axis_index("c")

        @pl.when(sid == 0)
        def _():
            pltpu.sync_copy(table_hbm, tab_sh)

        pltpu.sync_copy(idx_hbm.at[pl.ds(wid * ids_per_w, ids_per_w)], idx_v)
        plsc.subcore_barrier()

        n1 = (s // 8) * 8
        n2 = s - n1

        def fire(kk, rbuf, gsem):
            pltpu.async_copy(
                tab_sh.at[idx_v.at[pl.ds(kk * s, n1)]],
                rbuf.at[pl.ds(0, n1)], gsem)
            if n2:
                pltpu.async_copy(
                    tab_sh.at[idx_v.at[pl.ds(kk * s + n1, n2)]],
                    rbuf.at[pl.ds(n1, n2)], gsem)

        def wait_g(rbuf, gsem):
            pltpu.make_async_copy(
                tab_sh.at[idx_v.at[pl.ds(0, n1)]],
                rbuf.at[pl.ds(0, n1)], gsem).wait()
            if n2:
                pltpu.make_async_copy(
                    tab_sh.at[idx_v.at[pl.ds(0, n2)]],
                    rbuf.at[pl.ds(n1, n2)], gsem).wait()

        def write(kk, rbuf, wsem):
            pltpu.async_copy(
                rbuf, out_hbm.at[wid * batches_per_w + kk], wsem)

        def wait_w(rbuf, wsem):
            pltpu.make_async_copy(rbuf, out_hbm.at[0], wsem).wait()

        bufs = ((r0, g0, w0), (r1, g1, w1))
        fire(0, r0, g0)
        fire(1, r1, g1)

        def body(k2, carry):
            for bi, (rb, gs, ws) in enumerate(bufs):
                kk = 2 * k2 + bi
                wait_g(rb, gs)
                write(kk, rb, ws)
                wait_w(rb, ws)

                @pl.when(kk + 2 < batches_per_w)
                def _():
                    fire(kk + 2, rb, gs)

            return carry

        lax.fori_loop(0, batches_per_w // 2, body, 0)

    return k(table, ids_flat)


def kernel(token_ids, embedding_table):
    b, s = token_ids.shape
    v, d = embedding_table.shape
    assert b % NW == 0
    ids_flat = token_ids.reshape(-1).astype(jnp.int32)
    return _sc_gather(embedding_table, ids_flat, b, s, d)


# speedup vs baseline: 2.9168x; 2.0749x over previous
import functools

import jax
import jax.numpy as jnp
from jax import lax
from jax.experimental import pallas as pl
from jax.experimental.pallas import tpu as pltpu
from jax.experimental.pallas import tpu_sc as plsc

NC = 2
NS = 16
NW = NC * NS


def _sc_gather(table, ids_flat, b, s, d):
    v = table.shape[0]
    batches_per_w = b // NW
    ids_per_w = batches_per_w * s
    mesh = plsc.VectorSubcoreMesh(
        core_axis_name="c", subcore_axis_name="s", num_cores=NC,
        num_subcores=NS)

    @functools.partial(
        pl.kernel,
        mesh=mesh,
        compiler_params=pltpu.CompilerParams(use_tc_tiling_on_sc=False),
        out_type=jax.ShapeDtypeStruct((b, s, 128), jnp.float32),
        scratch_types=[
            pltpu.VMEM((ids_per_w,), jnp.int32),
            pltpu.VMEM((s, d), jnp.float32),
            pltpu.VMEM((s, d), jnp.float32),
            pltpu.VMEM_SHARED((v, d), jnp.float32),
            pltpu.SemaphoreType.DMA,
            pltpu.SemaphoreType.DMA,
            pltpu.SemaphoreType.DMA,
            pltpu.SemaphoreType.DMA,
        ],
    )
    def k(table_hbm, idx_hbm, out_hbm, idx_v, r0, r1, tab_sh,
          g0, g1, w0, w1):
        sid = lax.axis_index("s")
        wid = sid * NC + lax.axis_index("c")

        @pl.when(sid == 0)
        def _():
            pltpu.sync_copy(table_hbm, tab_sh)

        pltpu.sync_copy(idx_hbm.at[pl.ds(wid * ids_per_w, ids_per_w)], idx_v)
        plsc.subcore_barrier()

        n1 = (s // 8) * 8
        n2 = s - n1

        def fire(kk, rbuf, gsem):
            pltpu.async_copy(
                tab_sh.at[idx_v.at[pl.ds(kk * s, n1)]],
                rbuf.at[pl.ds(0, n1)], gsem)
            if n2:
                pltpu.async_copy(
                    tab_sh.at[idx_v.at[pl.ds(kk * s + n1, n2)]],
                    rbuf.at[pl.ds(n1, n2)], gsem)

        def wait_g(rbuf, gsem):
            pltpu.make_async_copy(
                tab_sh.at[idx_v.at[pl.ds(0, n1)]],
                rbuf.at[pl.ds(0, n1)], gsem).wait()
            if n2:
                pltpu.make_async_copy(
                    tab_sh.at[idx_v.at[pl.ds(0, n2)]],
                    rbuf.at[pl.ds(n1, n2)], gsem).wait()

        def write(kk, rbuf, wsem):
            pltpu.async_copy(
                rbuf, out_hbm.at[wid * batches_per_w + kk, :, pl.ds(0, d)],
                wsem)

        def wait_w(rbuf, wsem):
            pltpu.make_async_copy(
                rbuf, out_hbm.at[0, :, pl.ds(0, d)], wsem).wait()

        bufs = ((r0, g0, w0), (r1, g1, w1))
        fire(0, r0, g0)
        fire(1, r1, g1)

        def body(k2, carry):
            for bi, (rb, gs, ws) in enumerate(bufs):
                kk = 2 * k2 + bi
                wait_g(rb, gs)
                write(kk, rb, ws)
                wait_w(rb, ws)

                @pl.when(kk + 2 < batches_per_w)
                def _():
                    fire(kk + 2, rb, gs)

            return carry

        lax.fori_loop(0, batches_per_w // 2, body, 0)

    return k(table, ids_flat)


def kernel(token_ids, embedding_table):
    b, s = token_ids.shape
    v, d = embedding_table.shape
    assert b % NW == 0
    ids_flat = token_ids.reshape(-1).astype(jnp.int32)
    out = _sc_gather(embedding_table, ids_flat, b, s, d)
    return out[:, :, :d]


# speedup vs baseline: 2.9188x; 1.0007x over previous
import functools

import jax
import jax.numpy as jnp
from jax import lax
from jax.experimental import pallas as pl
from jax.experimental.pallas import tpu as pltpu
from jax.experimental.pallas import tpu_sc as plsc

NC = 2
NS = 16
NW = NC * NS
NBUF = 4
FIRE_AHEAD = 2


def _sc_gather(table, ids2, b, s, d):
    v = table.shape[0]
    batches_per_w = b // NW
    mesh = plsc.VectorSubcoreMesh(
        core_axis_name="c", subcore_axis_name="s", num_cores=NC,
        num_subcores=NS)

    @functools.partial(
        pl.kernel,
        mesh=mesh,
        compiler_params=pltpu.CompilerParams(use_tc_tiling_on_sc=False),
        out_type=jax.ShapeDtypeStruct((b, s, 128), jnp.float32),
        scratch_types=[
            pltpu.VMEM((batches_per_w, s), jnp.int32),
            [pltpu.VMEM((s, d), jnp.float32)] * NBUF,
            pltpu.VMEM_SHARED((v, d), jnp.float32),
            [pltpu.SemaphoreType.DMA] * NBUF,
            [pltpu.SemaphoreType.DMA] * NBUF,
        ],
    )
    def k(table_hbm, idx_hbm, out_hbm, idx_v, rbufs, tab_sh, gsems, wsems):
        sid = lax.axis_index("s")
        wid = sid * NC + lax.axis_index("c")

        @pl.when(sid == 0)
        def _():
            pltpu.sync_copy(table_hbm, tab_sh)

        pltpu.sync_copy(idx_hbm.at[pl.ds(wid * batches_per_w,
                                         batches_per_w)], idx_v)
        plsc.subcore_barrier()

        n1 = (s // 8) * 8
        n2 = s - n1

        def fire(kk, r):
            pltpu.async_copy(
                tab_sh.at[idx_v.at[kk, pl.ds(0, n1)]],
                rbufs[r].at[pl.ds(0, n1)], gsems[r])
            if n2:
                pltpu.async_copy(
                    tab_sh.at[idx_v.at[kk, pl.ds(n1, n2)]],
                    rbufs[r].at[pl.ds(n1, n2)], gsems[r])

        def wait_g(r):
            pltpu.make_async_copy(
                tab_sh.at[idx_v.at[0, pl.ds(0, n1)]],
                rbufs[r].at[pl.ds(0, n1)], gsems[r]).wait()
            if n2:
                pltpu.make_async_copy(
                    tab_sh.at[idx_v.at[0, pl.ds(0, n2)]],
                    rbufs[r].at[pl.ds(n1, n2)], gsems[r]).wait()

        def write(kk, r):
            pltpu.async_copy(
                rbufs[r].at[:, pl.ds(0, d)],
                out_hbm.at[wid * batches_per_w + kk, :, pl.ds(0, d)],
                wsems[r])

        def wait_w(r):
            pltpu.make_async_copy(
                rbufs[r].at[:, pl.ds(0, d)],
                out_hbm.at[0, :, pl.ds(0, d)], wsems[r]).wait()

        for c in range(FIRE_AHEAD):
            fire(c, c % NBUF)

        def body(c2, carry):
            for u in range(NBUF):
                c = NBUF * c2 + u
                r = u
                wait_g(r)
                write(c, r)
                rn = (u + FIRE_AHEAD) % NBUF

                @pl.when(c + FIRE_AHEAD < batches_per_w)
                def _():
                    @pl.when(c - (NBUF - FIRE_AHEAD) >= 0)
                    def _():
                        wait_w(rn)

                    fire(c + FIRE_AHEAD, rn)

            return carry

        lax.fori_loop(0, batches_per_w // NBUF, body, 0)
        for r in range(NBUF):
            wait_w(r)

    return k(table, ids2)


def kernel(token_ids, embedding_table):
    b, s = token_ids.shape
    v, d = embedding_table.shape
    assert b % (NW * NBUF) == 0
    ids2 = token_ids.astype(jnp.int32)
    out = _sc_gather(embedding_table, ids2, b, s, d)
    return out[:, :, :d]
